# Initial kernel scaffold; baseline (speedup 1.0000x reference)
import jax
import jax.numpy as jnp
from jax import lax
from jax.experimental import pallas as pl
from jax.experimental.pallas import tpu as pltpu
from jax.experimental.pallas import tpu_sc as plsc

N = 50000
E = 800000
HID = 64

NTILES = 16
HALF = N // 2
RPT = 1600
ACC_ROWS = NTILES * RPT
EP = 819200
PAD = EP - E
EPT = EP // NTILES
CH = 12800
NCH = EPT // CH
WIN = 128
NB = 4
WPC = CH // WIN
NGR = WPC // NB

_MESH = plsc.VectorSubcoreMesh(core_axis_name="c", subcore_axis_name="s")


def _calc_sidx(dst_st, sidx, b, wo, lo):
    for j in range(8):
        d = dst_st[pl.ds(wo + j * 16, 16)]
        m = (d >= lo) & (d < lo + HALF)
        trash = lax.iota(jnp.int32, 16) + (HALF + j * 16)
        sidx[b, pl.ds(j * 16, 16)] = jnp.where(m, d - lo, trash)


def _zero_rows(buf, nrows):
    zero16 = jnp.zeros((16,), jnp.float32)
    ncol = buf.shape[1]

    def body(r, _):
        for j in range(ncol // 16):
            buf[r, pl.ds(j * 16, 16)] = zero16
        return 0

    lax.fori_loop(0, nrows, body, 0)


def _copy_out(acc, out_h, t, lo):
    r0 = t * RPT

    @pl.when(t < NTILES - 1)
    def _():
        pltpu.sync_copy(acc.at[pl.ds(r0, RPT), :], out_h.at[pl.ds(lo + r0, RPT), :])

    @pl.when(t == NTILES - 1)
    def _():
        last = HALF - (NTILES - 1) * RPT
        pltpu.sync_copy(acc.at[pl.ds(r0, last), :],
                        out_h.at[pl.ds(lo + r0, last), :])


def _deg_body(dst_h, deg_h, acc, dst_st, ones, sidx, zbuf, ssem):
    c = lax.axis_index("c")
    t = lax.axis_index("s")
    lo = c * HALF

    one16 = jnp.full((16,), 1.0, jnp.float32)
    for r in range(WIN):
        ones[r, pl.ds(0, 16)] = one16
    _zero_rows(zbuf, 160)
    for q in range(RPT // 160):
        pltpu.sync_copy(zbuf, acc.at[pl.ds(t * RPT + q * 160, 160), :])
    plsc.subcore_barrier()

    def fire(b):
        pltpu.async_copy(ones, acc.at[sidx.at[b]], ssem.at[b], add=True)

    def drain(b):
        pltpu.make_async_copy(ones, acc.at[sidx.at[b]], ssem.at[b]).wait()

    for ch in range(NCH):
        ce = t * EPT + ch * CH
        pltpu.sync_copy(dst_h.at[pl.ds(ce, CH)], dst_st)
        for b in range(NB):
            _calc_sidx(dst_st, sidx, b, b * WIN, lo)
            fire(b)

        def grp(g, _):
            for b in range(NB):
                drain(b)
                wo = ((g + 1) * NB + b) * WIN
                _calc_sidx(dst_st, sidx, b, wo, lo)
                fire(b)
            return 0

        lax.fori_loop(0, NGR - 1, grp, 0)
        for b in range(NB):
            drain(b)

    plsc.subcore_barrier()
    _copy_out(acc, deg_h, t, lo)


def _scatter_body(src_h, dst_h, y_h, z_h,
                  acc, src_st, dst_st, rows, sidx, zbuf, gsem, ssem):
    c = lax.axis_index("c")
    t = lax.axis_index("s")
    lo = c * HALF

    _zero_rows(zbuf, 160)
    for q in range(RPT // 160):
        pltpu.sync_copy(zbuf, acc.at[pl.ds(t * RPT + q * 160, 160), :])
    plsc.subcore_barrier()

    def fire_gather(b, wo):
        pltpu.async_copy(y_h.at[src_st.at[pl.ds(wo, WIN)]], rows.at[b],
                         gsem.at[b])

    def wait_gather(b):
        pltpu.make_async_copy(y_h.at[pl.ds(0, WIN), :], rows.at[b],
                              gsem.at[b]).wait()

    def fire_scatter(b):
        pltpu.async_copy(rows.at[b], acc.at[sidx.at[b]], ssem.at[b], add=True)

    def wait_scatter(b):
        pltpu.make_async_copy(rows.at[b], acc.at[sidx.at[b]],
                              ssem.at[b]).wait()

    for ch in range(NCH):
        ce = t * EPT + ch * CH
        pltpu.sync_copy(src_h.at[pl.ds(ce, CH)], src_st)
        pltpu.sync_copy(dst_h.at[pl.ds(ce, CH)], dst_st)
        for b in range(NB):
            fire_gather(b, b * WIN)

        def grp(g, _):
            for b in range(NB):
                wo = (g * NB + b) * WIN
                wait_gather(b)
                _calc_sidx(dst_st, sidx, b, wo, lo)
                fire_scatter(b)
            for b in range(NB):
                wait_scatter(b)

                @pl.when(g < NGR - 1)
                def _():
                    fire_gather(b, ((g + 1) * NB + b) * WIN)
            return 0

        lax.fori_loop(0, NGR, grp, 0)

    plsc.subcore_barrier()
    _copy_out(acc, z_h, t, lo)


_deg_call = pl.kernel(
    _deg_body,
    out_type=jax.ShapeDtypeStruct((N, 16), jnp.float32),
    mesh=_MESH,
    scratch_types=[
        pltpu.VMEM_SHARED((ACC_ROWS, 16), jnp.float32),
        pltpu.VMEM((CH,), jnp.int32),
        pltpu.VMEM((WIN, 16), jnp.float32),
        pltpu.VMEM((NB, WIN), jnp.int32),
        pltpu.VMEM((160, 16), jnp.float32),
        pltpu.SemaphoreType.DMA((NB,)),
    ],
)

_scatter_call = pl.kernel(
    _scatter_body,
    out_type=jax.ShapeDtypeStruct((N, HID), jnp.float32),
    mesh=_MESH,
    scratch_types=[
        pltpu.VMEM_SHARED((ACC_ROWS, HID), jnp.float32),
        pltpu.VMEM((CH,), jnp.int32),
        pltpu.VMEM((CH,), jnp.int32),
        pltpu.VMEM((NB, WIN, HID), jnp.float32),
        pltpu.VMEM((NB, WIN), jnp.int32),
        pltpu.VMEM((160, HID), jnp.float32),
        pltpu.SemaphoreType.DMA((NB,)),
        pltpu.SemaphoreType.DMA((NB,)),
    ],
)


BN = 2000
GRID = N // BN
_P = lax.Precision.HIGHEST


def _k0_body(x_ref, mk_ref, dg_ref, w0_ref, b0_ref, wr_ref, br_ref,
             y_ref, base_ref, dis_ref):
    deg = dg_ref[:, 0:1] + 1.0
    dis = lax.rsqrt(deg)
    h0 = x_ref[...] * mk_ref[...]
    u0 = jnp.dot(h0, w0_ref[...], precision=_P)
    y0 = dis * u0
    res = jnp.dot(h0, wr_ref[...], precision=_P) + br_ref[...]
    y_ref[...] = y0
    base_ref[...] = res + b0_ref[...] + dis * y0
    dis_ref[...] = dis


def _kmid_body(z_ref, base_ref, dis_ref, w_ref, b_ref, y_ref, nbase_ref):
    dis = dis_ref[...]
    h = jnp.maximum(dis * z_ref[...] + base_ref[...], 0.0)
    u = jnp.dot(h, w_ref[...], precision=_P)
    y = dis * u
    y_ref[...] = y
    nbase_ref[...] = h + dis * y + b_ref[...]


def _k3_body(z_ref, base_ref, dis_ref, w_ref, b_ref, o_ref):
    dis = dis_ref[...]
    h = jnp.maximum(dis * z_ref[...] + base_ref[...], 0.0)
    o_ref[...] = jnp.dot(h, w_ref[...], precision=_P) + b_ref[...]


def _row_spec(w):
    return pl.BlockSpec((BN, w), lambda i: (i, 0))


def _full_spec(r, c):
    return pl.BlockSpec((r, c), lambda i: (0, 0))


_k0_call = pl.pallas_call(
    _k0_body,
    grid=(GRID,),
    in_specs=[_row_spec(4), _row_spec(1), _row_spec(16),
              _full_spec(4, HID), _full_spec(1, HID),
              _full_spec(4, HID), _full_spec(1, HID)],
    out_specs=[_row_spec(HID), _row_spec(HID), _row_spec(1)],
    out_shape=[jax.ShapeDtypeStruct((N, HID), jnp.float32),
               jax.ShapeDtypeStruct((N, HID), jnp.float32),
               jax.ShapeDtypeStruct((N, 1), jnp.float32)],
)

_kmid_call = pl.pallas_call(
    _kmid_body,
    grid=(GRID,),
    in_specs=[_row_spec(HID), _row_spec(HID), _row_spec(1),
              _full_spec(HID, HID), _full_spec(1, HID)],
    out_specs=[_row_spec(HID), _row_spec(HID)],
    out_shape=[jax.ShapeDtypeStruct((N, HID), jnp.float32),
               jax.ShapeDtypeStruct((N, HID), jnp.float32)],
)

_k3_call = pl.pallas_call(
    _k3_body,
    grid=(GRID,),
    in_specs=[_row_spec(HID), _row_spec(HID), _row_spec(1),
              _full_spec(HID, HID), _full_spec(1, HID)],
    out_specs=_row_spec(HID),
    out_shape=jax.ShapeDtypeStruct((N, HID), jnp.float32),
)


@jax.jit
def kernel(x, edge_index, hidden_mask, W0, b0, Wr0, br0, W1, b1, W2, b2,
           Wf, bf):
    src_p = jnp.concatenate(
        [edge_index[0], (jnp.arange(PAD, dtype=jnp.int32) * 37) % N])
    dst_p = jnp.concatenate(
        [edge_index[1], jnp.full((PAD,), jnp.int32(2 ** 30))])
    maskf = hidden_mask.astype(jnp.float32)[:, None]

    deg16 = _deg_call(dst_p)
    y0, base0, dis = _k0_call(x, maskf, deg16, W0, b0[None, :], Wr0,
                              br0[None, :])
    z0 = _scatter_call(src_p, dst_p, y0)
    y1, base1 = _kmid_call(z0, base0, dis, W1, b1[None, :])
    z1 = _scatter_call(src_p, dst_p, y1)
    y2, base2 = _kmid_call(z1, base1, dis, W2, b2[None, :])
    z2 = _scatter_call(src_p, dst_p, y2)
    x_out = _k3_call(z2, base2, dis, Wf, bf[None, :])
    return (x_out, hidden_mask)


# speedup vs baseline: 17.5467x; 17.5467x over previous
import jax
import jax.numpy as jnp
from jax import lax
from jax.experimental import pallas as pl
from jax.experimental.pallas import tpu as pltpu
from jax.experimental.pallas import tpu_sc as plsc

N = 50000
E = 800000
HID = 64

NTILES = 16
HALF = N // 2
RPT = 1600
ACC_ROWS = NTILES * RPT
EP = 819200
PAD = EP - E
EPT = EP // NTILES
CH = 2560
NCH = EPT // CH
WIN = 128
NB = 2
WPC = CH // WIN
NGR = WPC // NB
ZR = 80

_MESH = plsc.VectorSubcoreMesh(core_axis_name="c", subcore_axis_name="s")


def _calc_sidx(dst_st, sidx, b, wo, lo):
    for j in range(8):
        d = dst_st[pl.ds(wo + j * 16, 16)]
        m = (d >= lo) & (d < lo + HALF)
        trash = lax.iota(jnp.int32, 16) + (HALF + j * 16)
        sidx[b, pl.ds(j * 16, 16)] = jnp.where(m, d - lo, trash)


def _zero_rows(buf, nrows):
    zero16 = jnp.zeros((16,), jnp.float32)
    ncol = buf.shape[1]

    def body(r, _):
        for j in range(ncol // 16):
            buf[r, pl.ds(j * 16, 16)] = zero16
        return 0

    lax.fori_loop(0, nrows, body, 0)


def _copy_out(acc, out_h, t, lo):
    r0 = t * RPT

    @pl.when(t < NTILES - 1)
    def _():
        pltpu.sync_copy(acc.at[pl.ds(r0, RPT), :], out_h.at[pl.ds(lo + r0, RPT), :])

    @pl.when(t == NTILES - 1)
    def _():
        last = HALF - (NTILES - 1) * RPT
        pltpu.sync_copy(acc.at[pl.ds(r0, last), :],
                        out_h.at[pl.ds(lo + r0, last), :])


def _deg_body(dst_h, deg_h, acc, dst_st, ones, sidx, zbuf, ssem):
    c = lax.axis_index("c")
    t = lax.axis_index("s")
    lo = c * HALF

    one16 = jnp.full((16,), 1.0, jnp.float32)
    for r in range(WIN):
        ones[r, pl.ds(0, 16)] = one16
    _zero_rows(zbuf, ZR)
    for q in range(RPT // ZR):
        pltpu.sync_copy(zbuf, acc.at[pl.ds(t * RPT + q * ZR, ZR), :])
    plsc.subcore_barrier()

    def fire(b):
        pltpu.async_copy(ones, acc.at[sidx.at[b]], ssem.at[b], add=True)

    def drain(b):
        pltpu.make_async_copy(ones, acc.at[sidx.at[b]], ssem.at[b]).wait()

    for ch in range(NCH):
        ce = t * EPT + ch * CH
        pltpu.sync_copy(dst_h.at[pl.ds(ce, CH)], dst_st)
        for b in range(NB):
            _calc_sidx(dst_st, sidx, b, b * WIN, lo)
            fire(b)

        def grp(g, _):
            for b in range(NB):
                drain(b)
                wo = ((g + 1) * NB + b) * WIN
                _calc_sidx(dst_st, sidx, b, wo, lo)
                fire(b)
            return 0

        lax.fori_loop(0, NGR - 1, grp, 0)
        for b in range(NB):
            drain(b)

    plsc.subcore_barrier()
    _copy_out(acc, deg_h, t, lo)


def _scatter_body(src_h, dst_h, y_h, z_h,
                  acc, src_st, dst_st, rows, sidx, zbuf, gsem, ssem):
    c = lax.axis_index("c")
    t = lax.axis_index("s")
    lo = c * HALF

    _zero_rows(zbuf, ZR)
    for q in range(RPT // ZR):
        pltpu.sync_copy(zbuf, acc.at[pl.ds(t * RPT + q * ZR, ZR), :])
    plsc.subcore_barrier()

    def fire_gather(b, wo):
        pltpu.async_copy(y_h.at[src_st.at[pl.ds(wo, WIN)]], rows.at[b],
                         gsem.at[b])

    def wait_gather(b):
        pltpu.make_async_copy(y_h.at[pl.ds(0, WIN), :], rows.at[b],
                              gsem.at[b]).wait()

    def fire_scatter(b):
        pltpu.async_copy(rows.at[b], acc.at[sidx.at[b]], ssem.at[b], add=True)

    def wait_scatter(b):
        pltpu.make_async_copy(rows.at[b], acc.at[sidx.at[b]],
                              ssem.at[b]).wait()

    for ch in range(NCH):
        ce = t * EPT + ch * CH
        pltpu.sync_copy(src_h.at[pl.ds(ce, CH)], src_st)
        pltpu.sync_copy(dst_h.at[pl.ds(ce, CH)], dst_st)
        for b in range(NB):
            fire_gather(b, b * WIN)

        def grp(g, _):
            for b in range(NB):
                wo = (g * NB + b) * WIN
                wait_gather(b)
                _calc_sidx(dst_st, sidx, b, wo, lo)
                fire_scatter(b)
            for b in range(NB):
                wait_scatter(b)

                @pl.when(g < NGR - 1)
                def _():
                    fire_gather(b, ((g + 1) * NB + b) * WIN)
            return 0

        lax.fori_loop(0, NGR, grp, 0)

    plsc.subcore_barrier()
    _copy_out(acc, z_h, t, lo)


_SC_PARAMS = pltpu.CompilerParams(use_tc_tiling_on_sc=False)

_deg_call = pl.kernel(
    _deg_body,
    out_type=jax.ShapeDtypeStruct((N, 16), jnp.float32),
    mesh=_MESH,
    compiler_params=_SC_PARAMS,
    scratch_types=[
        pltpu.VMEM_SHARED((ACC_ROWS, 16), jnp.float32),
        pltpu.VMEM((CH,), jnp.int32),
        pltpu.VMEM((WIN, 16), jnp.float32),
        pltpu.VMEM((NB, WIN), jnp.int32),
        pltpu.VMEM((ZR, 16), jnp.float32),
        pltpu.SemaphoreType.DMA((NB,)),
    ],
)

_scatter_call = pl.kernel(
    _scatter_body,
    out_type=jax.ShapeDtypeStruct((N, HID), jnp.float32),
    mesh=_MESH,
    compiler_params=_SC_PARAMS,
    scratch_types=[
        pltpu.VMEM_SHARED((ACC_ROWS, HID), jnp.float32),
        pltpu.VMEM((CH,), jnp.int32),
        pltpu.VMEM((CH,), jnp.int32),
        pltpu.VMEM((NB, WIN, HID), jnp.float32),
        pltpu.VMEM((NB, WIN), jnp.int32),
        pltpu.VMEM((ZR, HID), jnp.float32),
        pltpu.SemaphoreType.DMA((NB,)),
        pltpu.SemaphoreType.DMA((NB,)),
    ],
)


BN = 2000
GRID = N // BN
_P = lax.Precision.HIGHEST


def _k0_body(x_ref, mk_ref, dg_ref, w0_ref, b0_ref, wr_ref, br_ref,
             y_ref, base_ref, dis_ref):
    deg = dg_ref[:, 0:1] + 1.0
    dis = lax.rsqrt(deg)
    h0 = x_ref[...] * mk_ref[...]
    u0 = jnp.dot(h0, w0_ref[...], precision=_P)
    y0 = dis * u0
    res = jnp.dot(h0, wr_ref[...], precision=_P) + br_ref[...]
    y_ref[...] = y0
    base_ref[...] = res + b0_ref[...] + dis * y0
    dis_ref[...] = dis


def _kmid_body(z_ref, base_ref, dis_ref, w_ref, b_ref, y_ref, nbase_ref):
    dis = dis_ref[...]
    h = jnp.maximum(dis * z_ref[...] + base_ref[...], 0.0)
    u = jnp.dot(h, w_ref[...], precision=_P)
    y = dis * u
    y_ref[...] = y
    nbase_ref[...] = h + dis * y + b_ref[...]


def _k3_body(z_ref, base_ref, dis_ref, w_ref, b_ref, o_ref):
    dis = dis_ref[...]
    h = jnp.maximum(dis * z_ref[...] + base_ref[...], 0.0)
    o_ref[...] = jnp.dot(h, w_ref[...], precision=_P) + b_ref[...]


def _row_spec(w):
    return pl.BlockSpec((BN, w), lambda i: (i, 0))


def _full_spec(r, c):
    return pl.BlockSpec((r, c), lambda i: (0, 0))


_k0_call = pl.pallas_call(
    _k0_body,
    grid=(GRID,),
    in_specs=[_row_spec(4), _row_spec(1), _row_spec(16),
              _full_spec(4, HID), _full_spec(1, HID),
              _full_spec(4, HID), _full_spec(1, HID)],
    out_specs=[_row_spec(HID), _row_spec(HID), _row_spec(1)],
    out_shape=[jax.ShapeDtypeStruct((N, HID), jnp.float32),
               jax.ShapeDtypeStruct((N, HID), jnp.float32),
               jax.ShapeDtypeStruct((N, 1), jnp.float32)],
)

_kmid_call = pl.pallas_call(
    _kmid_body,
    grid=(GRID,),
    in_specs=[_row_spec(HID), _row_spec(HID), _row_spec(1),
              _full_spec(HID, HID), _full_spec(1, HID)],
    out_specs=[_row_spec(HID), _row_spec(HID)],
    out_shape=[jax.ShapeDtypeStruct((N, HID), jnp.float32),
               jax.ShapeDtypeStruct((N, HID), jnp.float32)],
)

_k3_call = pl.pallas_call(
    _k3_body,
    grid=(GRID,),
    in_specs=[_row_spec(HID), _row_spec(HID), _row_spec(1),
              _full_spec(HID, HID), _full_spec(1, HID)],
    out_specs=_row_spec(HID),
    out_shape=jax.ShapeDtypeStruct((N, HID), jnp.float32),
)


@jax.jit
def kernel(x, edge_index, hidden_mask, W0, b0, Wr0, br0, W1, b1, W2, b2,
           Wf, bf):
    src_p = jnp.concatenate(
        [edge_index[0], (jnp.arange(PAD, dtype=jnp.int32) * 37) % N])
    dst_p = jnp.concatenate(
        [edge_index[1], jnp.full((PAD,), jnp.int32(2 ** 30))])
    maskf = hidden_mask.astype(jnp.float32)[:, None]

    deg16 = _deg_call(dst_p)
    y0, base0, dis = _k0_call(x, maskf, deg16, W0, b0[None, :], Wr0,
                              br0[None, :])
    z0 = _scatter_call(src_p, dst_p, y0)
    y1, base1 = _kmid_call(z0, base0, dis, W1, b1[None, :])
    z1 = _scatter_call(src_p, dst_p, y1)
    y2, base2 = _kmid_call(z1, base1, dis, W2, b2[None, :])
    z2 = _scatter_call(src_p, dst_p, y2)
    x_out = _k3_call(z2, base2, dis, Wf, bf[None, :])
    return (x_out, hidden_mask)
